# baseline (speedup 1.0000x reference)
import jax
import jax.numpy as jnp
from jax.experimental import pallas as pl
from jax.experimental.pallas import tpu as pltpu


def _leaky(x, s):
    return jnp.where(x > 0, x, s * x)


def _gatv2(x, src, dst, Wl, bl, Wr, br, att, bias, heads, out_ch, concat):
    n = x.shape[0]
    xl = (x @ Wl + bl).reshape(n, heads, out_ch)
    xr = (x @ Wr + br).reshape(n, heads, out_ch)
    e = _leaky(xl[src] + xr[dst], 0.2)
    alpha = (e * att[None, :, :]).sum(-1)
    amax = jax.ops.segment_max(alpha, dst, num_segments=n)
    amax = jnp.where(jnp.isfinite(amax), amax, 0.0)
    ex = jnp.exp(alpha - amax[dst])
    den = jax.ops.segment_sum(ex, dst, num_segments=n)
    a = ex / (den[dst] + 1e-16)
    out = jax.ops.segment_sum(xl[src] * a[:, :, None], dst, num_segments=n)
    if concat:
        out = out.reshape(n, heads * out_ch)
    else:
        out = out.mean(axis=1)
    return out + bias


def _final_mm_kernel(p_ref, w_ref, b_ref, o_ref):
    p = _leaky(p_ref[...], 0.01)
    o_ref[...] = p @ w_ref[...] + b_ref[...]


def kernel(x, edge_index, batch, Wl1, bl1, Wr1, br1, att1, bias1, Wl2, bl2, Wr2, br2, att2, bias2, Wl3, bl3, Wr3, br3, att3, bias3, Wh, bh):
    src, dst = edge_index[0], edge_index[1]
    h = _gatv2(x, src, dst, Wl1, bl1, Wr1, br1, att1, bias1, 8, 16, True)
    h = _leaky(h, 0.01)
    h = _gatv2(h, src, dst, Wl2, bl2, Wr2, br2, att2, bias2, 8, 16, True)
    h = _leaky(h, 0.01)
    h = _gatv2(h, src, dst, Wl3, bl3, Wr3, br3, att3, bias3, 1, 128, False)
    G = 256
    pmin = jax.ops.segment_min(h, batch, num_segments=G)
    pmax = jax.ops.segment_max(h, batch, num_segments=G)
    psum = jax.ops.segment_sum(h, batch, num_segments=G)
    cnt = jax.ops.segment_sum(jnp.ones((h.shape[0], 1), h.dtype), batch, num_segments=G)
    pmean = psum / jnp.maximum(cnt, 1.0)
    pooled = jnp.concatenate([pmin, pmax, pmean, psum], axis=1)
    out = pl.pallas_call(
        _final_mm_kernel,
        out_shape=jax.ShapeDtypeStruct((G, Wh.shape[1]), jnp.float32),
    )(pooled, Wh, bh[None, :])
    return out


# speedup vs baseline: 7.2101x; 7.2101x over previous
import functools

import jax
import jax.numpy as jnp
from jax import lax
from jax.experimental import pallas as pl
from jax.experimental.pallas import tpu as pltpu
from jax.experimental.pallas import tpu_sc as plsc

_SC_PARAMS = pltpu.CompilerParams(
    needs_layout_passes=False, use_tc_tiling_on_sc=False)

NC = 2
NS = 16
NT = NC * NS

CHB = 12
CH = 1 << CHB
PADQ = 2048
NB = 13
NTOT = NB * CH
ROWBLK = 512
F32 = jnp.float32
I32 = jnp.int32


def _leaky(x, s):
    return jnp.where(x > 0, x, s * x)


def _iota16():
    return lax.iota(I32, 16)


def _take16(v, idx):
    return lax.gather(
        v, idx[:, None],
        dimension_numbers=lax.GatherDimensionNumbers(
            offset_dims=(), collapsed_slice_dims=(0,), start_index_map=(0,)),
        slice_sizes=(1,), mode=lax.GatherScatterMode.PROMISE_IN_BOUNDS)


def _extract(v, k):
    return jnp.sum(jnp.where(_iota16() == k, v, 0))


def _wid():
    return lax.axis_index("s") * NC + lax.axis_index("c")


def _mesh():
    return plsc.VectorSubcoreMesh(core_axis_name="c", subcore_axis_name="s")


def _hist_body(ept, nfull, tail, dst_hbm, hist_hbm, dbuf, hv_ref):
    wid = _wid()
    base = wid * ept
    iota = _iota16()

    def vreg_update(d, valid, hv):
        b = jnp.clip(lax.shift_right_logical(d, CHB), 0, 15)
        for b_id in range(NB):
            m = jnp.logical_and(b == b_id, valid)
            pc = jnp.sum(jnp.where(m, 1, 0))
            hv = hv + jnp.where(iota == b_id, pc, 0)
        return hv

    def block(i, hv):
        pltpu.sync_copy(dst_hbm.at[pl.ds(base + i * 2048, 2048)], dbuf)

        def inner(j, hv):
            return vreg_update(dbuf[pl.ds(16 * j, 16)], iota >= 0, hv)

        return lax.fori_loop(0, 128, inner, hv)

    hv = lax.fori_loop(0, nfull, block, jnp.zeros((16,), I32))
    if tail:
        tail8 = (tail + 7) // 8 * 8
        pltpu.sync_copy(dst_hbm.at[pl.ds(base + nfull * 2048, tail8)],
                        dbuf.at[pl.ds(0, tail8)])
        ntv = tail // 16

        def inner_t(j, hv):
            return vreg_update(dbuf[pl.ds(16 * j, 16)], iota >= 0, hv)

        hv = lax.fori_loop(0, ntv, inner_t, hv)
        rem = tail - ntv * 16
        if rem:
            hv = vreg_update(dbuf[pl.ds(16 * ntv, 16)], iota < rem, hv)
    hv_ref[...] = hv
    pltpu.sync_copy(hv_ref, hist_hbm.at[wid])


def _scat_body(ept, nfull, tail, eb, src_hbm, dst_hbm, hist_hbm,
               srcb_hbm, dstb_hbm, sbuf, dbuf, hbuf, posbuf):
    wid = _wid()
    base = wid * ept
    iota = _iota16()
    pltpu.sync_copy(hist_hbm, hbuf)
    tot = jnp.zeros((16,), I32)
    prior = jnp.zeros((16,), I32)
    for t in range(NT):
        row = hbuf[t]
        tot = tot + row
        prior = prior + jnp.where(wid > t, row, 0)
    sizes = jnp.bitwise_and(tot + (PADQ - 1), ~(PADQ - 1))
    bases = plsc.cumsum(sizes) - sizes
    startv = bases + prior

    def group(d, valid, cnts):
        b = jnp.clip(lax.shift_right_logical(d, CHB), 0, 15)
        pos = eb + iota
        newc = cnts
        for b_id in range(NB):
            m = jnp.logical_and(b == b_id, valid)
            mi = jnp.where(m, 1, 0)
            cs = plsc.cumsum(mi)
            rank = cs - mi
            pc = jnp.sum(mi)
            cb = _take16(startv + newc, jnp.full((16,), b_id, I32))
            pos = jnp.where(m, cb + rank, pos)
            newc = newc + jnp.where(iota == b_id, pc, 0)
        posbuf[...] = pos
        return newc

    def do_group(off, valid, cnts):
        cnts = group(dbuf[pl.ds(off, 16)], valid, cnts)
        pltpu.sync_copy(sbuf.at[pl.ds(off, 16)], srcb_hbm.at[posbuf])
        pltpu.sync_copy(dbuf.at[pl.ds(off, 16)], dstb_hbm.at[posbuf])
        return cnts

    def block(i, cnts):
        pltpu.sync_copy(dst_hbm.at[pl.ds(base + i * 2048, 2048)], dbuf)
        pltpu.sync_copy(src_hbm.at[pl.ds(base + i * 2048, 2048)], sbuf)

        def inner(j, cnts):
            return do_group(16 * j, iota >= 0, cnts)

        return lax.fori_loop(0, 128, inner, cnts)

    cnts = lax.fori_loop(0, nfull, block, jnp.zeros((16,), I32))
    if tail:
        tail8 = (tail + 7) // 8 * 8
        pltpu.sync_copy(dst_hbm.at[pl.ds(base + nfull * 2048, tail8)],
                        dbuf.at[pl.ds(0, tail8)])
        pltpu.sync_copy(src_hbm.at[pl.ds(base + nfull * 2048, tail8)],
                        sbuf.at[pl.ds(0, tail8)])
        ntv = tail // 16

        def inner_t(j, cnts):
            return do_group(16 * j, iota >= 0, cnts)

        cnts = lax.fori_loop(0, ntv, inner_t, cnts)
        rem = tail - ntv * 16
        if rem:
            cnts = do_group(16 * ntv, iota < rem, cnts)


def _bucket_edges(src, dst, eb):
    e = src.shape[0]
    ept = e // NT
    assert ept * NT == e
    nfull, tail = divmod(ept, 2048)

    hist = pl.kernel(
        functools.partial(_hist_body, ept, nfull, tail),
        out_type=jax.ShapeDtypeStruct((NT, 16), I32),
        mesh=_mesh(),
        compiler_params=_SC_PARAMS,
        scratch_types=[
            pltpu.VMEM((2048,), I32),
            pltpu.VMEM((16,), I32),
        ],
    )(dst)

    srcb, dstb = pl.kernel(
        functools.partial(_scat_body, ept, nfull, tail, eb),
        out_type=[
            jax.ShapeDtypeStruct((eb + 16,), I32),
            jax.ShapeDtypeStruct((eb + 16,), I32),
        ],
        mesh=_mesh(),
        compiler_params=_SC_PARAMS,
        scratch_types=[
            pltpu.VMEM((2048,), I32),
            pltpu.VMEM((2048,), I32),
            pltpu.VMEM((NT, 16), I32),
            pltpu.VMEM((16,), I32),
        ],
    )(src, dst, hist)
    return srcb, dstb, hist


def _passa_body(eb, npad, sum_heads,
                srcb, dstb, bctl, att_hbm,
                xl0, xl1, xl2, xl3, xr0, xr1, xr2, xr3,
                alphaT, amax_sc,
                srcv, dstv, xlb, xrb, albuf, attv, bctlv, amax_loc,
                tmp, tmp2, slots, sem):
    core = lax.axis_index("c")
    sid = lax.axis_index("s")
    wid = sid * NC + core
    iota = _iota16()
    xls = (xl0, xl1, xl2, xl3)
    xrs = (xr0, xr1, xr2, xr3)

    pltpu.sync_copy(bctl, bctlv)
    pltpu.sync_copy(att_hbm, attv)
    bstartv = bctlv[0]
    btotv = bctlv[1]
    sharev = bctlv[2]

    def bucket(k, _):
        b0 = pl.multiple_of(_extract(bstartv, k), PADQ)
        tot = _extract(btotv, k)
        share = pl.multiple_of(_extract(sharev, k), 64)
        mystart = pl.multiple_of(b0 + wid * share, 64)
        end_valid = b0 + tot
        ng = jnp.maximum(
            jnp.minimum((end_valid - mystart + 63) // 64, share // 64), 0)

        def zrow(h, _):
            def zcol(j, _):
                amax_loc[h, pl.ds(16 * j, 16)] = jnp.full((16,), -jnp.inf, F32)
                return 0
            lax.fori_loop(0, CH // 16, zcol, 0)
            return 0
        lax.fori_loop(0, 8, zrow, 0)

        def group(g, _):
            base = pl.multiple_of(mystart + g * 64, 64)
            pltpu.sync_copy(srcb.at[pl.ds(base, 64)], srcv)
            pltpu.sync_copy(dstb.at[pl.ds(base, 64)], dstv)
            for i in range(4):
                srcv[pl.ds(16 * i, 16)] = jnp.clip(
                    srcv[pl.ds(16 * i, 16)], 0, npad - 1)
                dstv[pl.ds(16 * i, 16)] = jnp.clip(
                    dstv[pl.ds(16 * i, 16)], 0, npad - 1)
            cps = []
            for s in range(4):
                cps.append(pltpu.async_copy(xls[s].at[srcv], xlb.at[s], sem))
                cps.append(pltpu.async_copy(xrs[s].at[dstv], xrb.at[s], sem))
            for cp in cps:
                cp.wait()

            for g4 in range(4):
                dst16 = dstv[pl.ds(16 * g4, 16)]
                valid = (base + 16 * g4) + iota < end_valid
                dstloc = jnp.clip(dst16 - k * CH, 0, CH - 1)

                def head_alpha(h, _):
                    att_h = attv[pl.ds(pl.multiple_of(16 * h, 16), 16)]
                    ev = 16 * g4 + iota

                    def chan(cc, acc):
                        attb = _take16(att_h, jnp.full((16,), cc, I32))
                        s = lax.shift_right_logical(h, 1)
                        c = (h & 1) * 16 + cc
                        xlc = plsc.load_gather(
                            xlb, [jnp.full((16,), s, I32), ev,
                                  jnp.full((16,), c, I32)])
                        xrc = plsc.load_gather(
                            xrb, [jnp.full((16,), s, I32), ev,
                                  jnp.full((16,), c, I32)])
                        z = xlc + xrc
                        return acc + attb * _leaky(z, 0.2)

                    acc = lax.fori_loop(0, 16, chan, jnp.zeros((16,), F32))
                    off = pl.multiple_of(64 * h + 16 * g4, 16)
                    albuf[pl.ds(off, 16)] = acc
                    return 0
                lax.fori_loop(0, 8, head_alpha, 0)

                if sum_heads:
                    tv = jnp.zeros((16,), F32)
                    for h in range(8):
                        tv = tv + albuf[pl.ds(64 * h + 16 * g4, 16)]
                    for h in range(8):
                        albuf[pl.ds(64 * h + 16 * g4, 16)] = tv

                def head_max(h, _):
                    off = pl.multiple_of(64 * h + 16 * g4, 16)
                    al = albuf[pl.ds(off, 16)]
                    hv = jnp.full((16,), h, I32)

                    def wbody(done):
                        cur = plsc.load_gather(amax_loc, [hv, dstloc])
                        need = jnp.logical_and(valid, al > cur)
                        plsc.store_scatter(amax_loc, [hv, dstloc],
                                           jnp.maximum(cur, al), mask=need)
                        cur2 = plsc.load_gather(amax_loc, [hv, dstloc])
                        bad = jnp.sum(jnp.where(
                            jnp.logical_and(valid, al > cur2), 1, 0))
                        return bad == 0
                    lax.while_loop(lambda d: jnp.logical_not(d), wbody, False)
                    return 0
                lax.fori_loop(0, 8, head_max, 0)

            for h in range(8):
                pltpu.sync_copy(albuf.at[pl.ds(64 * h, 64)],
                                alphaT.at[h, pl.ds(base, 64)])
            return 0
        lax.fori_loop(0, ng, group, 0)

        pltpu.sync_copy(amax_loc, slots.at[sid])
        plsc.subcore_barrier()
        mycol = sid * (CH // NS)
        pltpu.sync_copy(slots.at[0, :, pl.ds(mycol, CH // NS)], tmp)
        for j in range(1, NS):
            pltpu.sync_copy(slots.at[j, :, pl.ds(mycol, CH // NS)], tmp2)

            def mrow(h, _):
                def mcol(q, _):
                    tmp[h, pl.ds(16 * q, 16)] = jnp.maximum(
                        tmp[h, pl.ds(16 * q, 16)], tmp2[h, pl.ds(16 * q, 16)])
                    return 0
                lax.fori_loop(0, CH // NS // 16, mcol, 0)
                return 0
            lax.fori_loop(0, 8, mrow, 0)
        pltpu.sync_copy(tmp, amax_sc.at[core, k, :, pl.ds(mycol, CH // NS)])
        plsc.subcore_barrier()
        return 0
    lax.fori_loop(0, NB, bucket, 0)


def _run_passa(srcb, dstb, bctl, att_flat, slabs, eb, npad, sum_heads):
    return pl.kernel(
        functools.partial(_passa_body, eb, npad, sum_heads),
        out_type=[
            jax.ShapeDtypeStruct((8, eb), F32),
            jax.ShapeDtypeStruct((NC, NB, 8, CH), F32),
        ],
        mesh=_mesh(),
        compiler_params=_SC_PARAMS,
        scratch_types=[
            pltpu.VMEM((64,), I32),
            pltpu.VMEM((64,), I32),
            pltpu.VMEM((4, 64, 32), F32),
            pltpu.VMEM((4, 64, 32), F32),
            pltpu.VMEM((512,), F32),
            pltpu.VMEM((128,), F32),
            pltpu.VMEM((3, 16), I32),
            pltpu.VMEM((8, CH), F32),
            pltpu.VMEM((8, CH // NS), F32),
            pltpu.VMEM((8, CH // NS), F32),
            pltpu.VMEM_SHARED((NS, 8, CH), F32),
            pltpu.SemaphoreType.DMA,
        ],
    )(srcb, dstb, bctl, att_flat, *slabs)


def _passb_body(eb, npad,
                srcb, dstb, bctl, alphaT, amax_sc,
                xl0, xl1, xl2, xl3,
                accv_hbm, den_hbm,
                srcv, dstred, xlb, albuf, exb, rowsb, denb, bctlv,
                amax_loc, amax_tmp, zbuf, denz, accsp, densp, sem):
    core = lax.axis_index("c")
    sid = lax.axis_index("s")
    wid = sid * NC + core
    iota = _iota16()
    xls = (xl0, xl1, xl2, xl3)

    pltpu.sync_copy(bctl, bctlv)
    bstartv = bctlv[0]
    btotv = bctlv[1]
    sharev = bctlv[2]

    zvec = jnp.zeros((16,), F32)

    def zb(i, _):
        plsc.store_scatter(
            zbuf, [jnp.full((16,), i // 8, I32), 16 * (i % 8) + iota], zvec)
        return 0
    lax.fori_loop(0, 64 * 8, zb, 0)

    def zd(i, _):
        plsc.store_scatter(
            denz, [2 * i + lax.shift_right_logical(iota, 3), iota & 7], zvec)
        return 0
    lax.fori_loop(0, 16, zd, 0)

    def bucket(k, _):
        b0 = pl.multiple_of(_extract(bstartv, k), PADQ)
        tot = _extract(btotv, k)
        share = pl.multiple_of(_extract(sharev, k), 64)
        mystart = pl.multiple_of(b0 + wid * share, 64)
        end_valid = b0 + tot
        ng = jnp.maximum(
            jnp.minimum((end_valid - mystart + 63) // 64, share // 64), 0)

        pltpu.sync_copy(amax_sc.at[0, k], amax_loc)
        pltpu.sync_copy(amax_sc.at[1, k], amax_tmp)

        def mrow(h, _):
            def mcol(q, _):
                amax_loc[h, pl.ds(16 * q, 16)] = jnp.maximum(
                    amax_loc[h, pl.ds(16 * q, 16)],
                    amax_tmp[h, pl.ds(16 * q, 16)])
                return 0
            lax.fori_loop(0, CH // 16, mcol, 0)
            return 0
        lax.fori_loop(0, 8, mrow, 0)

        myrow = sid * (CH // NS)
        for q in range(CH // NS // 64):
            pltpu.sync_copy(zbuf, accsp.at[pl.ds(myrow + 64 * q, 64)])
        pltpu.sync_copy(denz, densp.at[pl.ds(myrow, 32)])
        for q in range(1, CH // NS // 32):
            pltpu.sync_copy(denz, densp.at[pl.ds(myrow + 32 * q, 32)])
        plsc.subcore_barrier()

        def group(g, _):
            base = pl.multiple_of(mystart + g * 64, 64)
            pltpu.sync_copy(srcb.at[pl.ds(base, 64)], srcv)
            pltpu.sync_copy(dstb.at[pl.ds(base, 64)], dstred)
            for i in range(4):
                srcv[pl.ds(16 * i, 16)] = jnp.clip(
                    srcv[pl.ds(16 * i, 16)], 0, npad - 1)
            cps = [pltpu.async_copy(xls[s].at[srcv], xlb.at[s], sem)
                   for s in range(4)]
            for h in range(8):
                pltpu.sync_copy(alphaT.at[h, pl.ds(base, 64)],
                                albuf.at[pl.ds(64 * h, 64)])
            for cp in cps:
                cp.wait()

            for g4 in range(4):
                dst16 = dstred[pl.ds(16 * g4, 16)]
                valid = (base + 16 * g4) + iota < end_valid
                dstloc = jnp.clip(dst16 - k * CH, 0, CH - 1)
                ev = 16 * g4 + iota

                def head_ex(h, _):
                    off = pl.multiple_of(64 * h + 16 * g4, 16)
                    al = albuf[pl.ds(off, 16)]
                    am = plsc.load_gather(
                        amax_loc, [jnp.full((16,), h, I32), dstloc])
                    ex = jnp.where(valid, jnp.exp(al - am), 0.0)
                    exb[pl.ds(off, 16)] = ex
                    plsc.store_scatter(denb, [ev, jnp.full((16,), h, I32)],
                                       ex)
                    return 0
                lax.fori_loop(0, 8, head_ex, 0)

                for s in range(4):
                    def chan(cc, _):
                        ch = 32 * s + cc
                        h = lax.shift_right_logical(ch, 4)
                        xlc = plsc.load_gather(
                            xlb, [jnp.full((16,), s, I32), ev,
                                  jnp.full((16,), cc, I32)])
                        exv = plsc.load_gather(
                            exb, [64 * h + 16 * g4 + iota])
                        plsc.store_scatter(
                            rowsb, [ev, jnp.full((16,), ch, I32)], xlc * exv)
                        return 0
                    lax.fori_loop(0, 32, chan, 0)

                dstred[pl.ds(16 * g4, 16)] = jnp.where(valid, dstloc, CH)

            pltpu.sync_copy(rowsb, accsp.at[dstred], add=True)
            pltpu.sync_copy(denb, densp.at[dstred], add=True)
            return 0
        lax.fori_loop(0, ng, group, 0)

        plsc.subcore_barrier()
        pltpu.sync_copy(accsp.at[pl.ds(myrow, CH // NS)],
                        accv_hbm.at[core, pl.ds(k * CH + myrow, CH // NS)])
        pltpu.sync_copy(densp.at[pl.ds(myrow, CH // NS)],
                        den_hbm.at[core, pl.ds(k * CH + myrow, CH // NS)])
        return 0
    lax.fori_loop(0, NB, bucket, 0)


def _run_passb(srcb, dstb, bctl, alphaT, amax_sc, xlslabs, eb, npad):
    return pl.kernel(
        functools.partial(_passb_body, eb, npad),
        out_type=[
            jax.ShapeDtypeStruct((NC, NTOT, 128), F32),
            jax.ShapeDtypeStruct((NC, NTOT, 8), F32),
        ],
        mesh=_mesh(),
        compiler_params=_SC_PARAMS,
        scratch_types=[
            pltpu.VMEM((64,), I32),
            pltpu.VMEM((64,), I32),
            pltpu.VMEM((4, 64, 32), F32),
            pltpu.VMEM((512,), F32),
            pltpu.VMEM((512,), F32),
            pltpu.VMEM((64, 128), F32),
            pltpu.VMEM((64, 8), F32),
            pltpu.VMEM((3, 16), I32),
            pltpu.VMEM((8, CH), F32),
            pltpu.VMEM((8, CH), F32),
            pltpu.VMEM((64, 128), F32),
            pltpu.VMEM((32, 8), F32),
            pltpu.VMEM_SHARED((CH + 1, 128), F32),
            pltpu.VMEM_SHARED((CH + 1, 8), F32),
            pltpu.SemaphoreType.DMA,
        ],
    )(srcb, dstb, bctl, alphaT, amax_sc, *xlslabs)


def _mm1_kernel(x_ref, w_ref, b_ref, *outs):
    y = jnp.dot(x_ref[...], w_ref[...],
                preferred_element_type=F32) + b_ref[...]
    for s in range(8):
        outs[s][...] = y[:, 32 * s:32 * s + 32]


def _mm23_kernel(leak, bp_ref, a0_ref, a1_ref, d0_ref, d1_ref, w_ref,
                 b_ref, *outs):
    a = a0_ref[0] + a1_ref[0]
    d8 = d0_ref[0] + d1_ref[0]
    dfull = jnp.repeat(d8, 16, axis=1)
    h = jnp.where(dfull > 0.5, a / dfull, 0.0) + bp_ref[...]
    if leak:
        h = _leaky(h, 0.01)
    y = jnp.dot(h, w_ref[...], preferred_element_type=F32) + b_ref[...]
    for s in range(8):
        outs[s][...] = y[:, 32 * s:32 * s + 32]


def _ep3_kernel(bp_ref, a0_ref, a1_ref, d0_ref, d1_ref, o_ref):
    a = a0_ref[0] + a1_ref[0]
    d8 = d0_ref[0] + d1_ref[0]
    dfull = jnp.repeat(d8, 16, axis=1)
    o_ref[...] = jnp.where(dfull > 0.5, a / dfull, 0.0) + bp_ref[...]


def _mm_slabs_1(xp, w, bcat, npad):
    grid = npad // ROWBLK
    outs = pl.pallas_call(
        _mm1_kernel,
        grid=(grid,),
        in_specs=[
            pl.BlockSpec((ROWBLK, 128), lambda i: (i, 0)),
            pl.BlockSpec((128, 256), lambda i: (0, 0)),
            pl.BlockSpec((1, 256), lambda i: (0, 0)),
        ],
        out_specs=[pl.BlockSpec((ROWBLK, 32), lambda i: (i, 0))] * 8,
        out_shape=[jax.ShapeDtypeStruct((npad, 32), F32)] * 8,
    )(xp, w, bcat)
    return outs


def _mm_slabs_23(accv, den, bias_prev, w, bcat, npad, leak=True):
    grid = npad // ROWBLK
    outs = pl.pallas_call(
        functools.partial(_mm23_kernel, leak),
        grid=(grid,),
        in_specs=[
            pl.BlockSpec((1, 128), lambda i: (0, 0)),
            pl.BlockSpec((1, ROWBLK, 128), lambda i: (0, i, 0)),
            pl.BlockSpec((1, ROWBLK, 128), lambda i: (1, i, 0)),
            pl.BlockSpec((1, ROWBLK, 8), lambda i: (0, i, 0)),
            pl.BlockSpec((1, ROWBLK, 8), lambda i: (1, i, 0)),
            pl.BlockSpec((128, 256), lambda i: (0, 0)),
            pl.BlockSpec((1, 256), lambda i: (0, 0)),
        ],
        out_specs=[pl.BlockSpec((ROWBLK, 32), lambda i: (i, 0))] * 8,
        out_shape=[jax.ShapeDtypeStruct((npad, 32), F32)] * 8,
    )(bias_prev, accv, accv, den, den, w, bcat)
    return outs


def _epilogue3(accv, den, bias_prev, npad):
    grid = npad // ROWBLK
    return pl.pallas_call(
        _ep3_kernel,
        grid=(grid,),
        in_specs=[
            pl.BlockSpec((1, 128), lambda i: (0, 0)),
            pl.BlockSpec((1, ROWBLK, 128), lambda i: (0, i, 0)),
            pl.BlockSpec((1, ROWBLK, 128), lambda i: (1, i, 0)),
            pl.BlockSpec((1, ROWBLK, 8), lambda i: (0, i, 0)),
            pl.BlockSpec((1, ROWBLK, 8), lambda i: (1, i, 0)),
        ],
        out_specs=pl.BlockSpec((ROWBLK, 128), lambda i: (i, 0)),
        out_shape=jax.ShapeDtypeStruct((npad, 128), F32),
    )(bias_prev, accv, accv, den, den)


def _bnd_body(n, share, batchp, starts_hbm, ends_hbm, bbuf, posv, valv):
    wid = _wid()
    base = wid * share
    iota = _iota16()
    bbuf[pl.ds(0, 16)] = jnp.full((16,), -1, I32)
    bbuf[pl.ds(share, 16)] = jnp.full((16,), -2, I32)

    @pl.when(wid > 0)
    def _():
        pltpu.sync_copy(batchp.at[pl.ds(base - 8, 8)], bbuf.at[pl.ds(0, 8)])

    pltpu.sync_copy(batchp.at[pl.ds(base, share)], bbuf.at[pl.ds(8, share)])

    @pl.when(wid < NT - 1)
    def _():
        pltpu.sync_copy(batchp.at[pl.ds(base + share, 8)],
                        bbuf.at[pl.ds(share + 8, 8)])

    def vreg(j, _):
        cur = bbuf[pl.ds(8 + 16 * j, 16)]
        prev = plsc.load_gather(bbuf, [7 + 16 * j + iota])
        nxt = plsc.load_gather(bbuf, [9 + 16 * j + iota])
        gpos = base + 16 * j + iota
        isst = jnp.logical_and(cur != prev, gpos < n)
        isen = jnp.logical_and(cur != nxt, gpos < n)
        posv[...] = jnp.where(isst, jnp.clip(cur, 0, 262), 263)
        valv[...] = gpos
        pltpu.sync_copy(valv, starts_hbm.at[posv])
        posv[...] = jnp.where(isen, jnp.clip(cur, 0, 262), 263)
        valv[...] = gpos + 1
        pltpu.sync_copy(valv, ends_hbm.at[posv])
        return 0
    lax.fori_loop(0, share // 16, vreg, 0)


def _pool_body(n, npad, h3, starts_hbm, ends_hbm, pooled_hbm,
               sbuf, ebuf, hchunk, rowbuf, sem):
    del npad, sem
    wid = _wid()
    iota = _iota16()
    pltpu.sync_copy(starts_hbm, sbuf)
    pltpu.sync_copy(ends_hbm, ebuf)

    for j in range(8):
        g = wid * 8 + j
        vidx = pl.multiple_of((g // 16) * 16, 16)
        lane = g % 16
        st = jnp.clip(_extract(sbuf[pl.ds(vidx, 16)], lane), 0, n)
        en = jnp.clip(_extract(ebuf[pl.ds(vidx, 16)], lane), st, n)
        cnt = en - st
        nchunks = (cnt + 63) // 64

        def chunk(c, carry):
            r0 = st + 64 * c
            pltpu.sync_copy(h3.at[pl.ds(r0, 64)], hchunk)

            def row(r, carry):
                mns, mxs, sms = carry
                ok = 64 * c + r < cnt
                nmns, nmxs, nsms = [], [], []
                for c8 in range(8):
                    v = plsc.load_gather(
                        hchunk, [jnp.full((16,), r, I32), 16 * c8 + iota])
                    nmns.append(jnp.where(ok, jnp.minimum(mns[c8], v),
                                          mns[c8]))
                    nmxs.append(jnp.where(ok, jnp.maximum(mxs[c8], v),
                                          mxs[c8]))
                    nsms.append(jnp.where(ok, sms[c8] + v, sms[c8]))
                return tuple(nmns), tuple(nmxs), tuple(nsms)
            return lax.fori_loop(0, 64, row, carry)

        init = (tuple(jnp.full((16,), jnp.inf, F32) for _ in range(8)),
                tuple(jnp.full((16,), -jnp.inf, F32) for _ in range(8)),
                tuple(jnp.zeros((16,), F32) for _ in range(8)))
        mns, mxs, sms = lax.fori_loop(0, nchunks, chunk, init)
        cntf = jnp.maximum(cnt.astype(F32), 1.0)
        for c8 in range(8):
            rowbuf[pl.ds(16 * c8, 16)] = mns[c8]
            rowbuf[pl.ds(128 + 16 * c8, 16)] = mxs[c8]
            rowbuf[pl.ds(256 + 16 * c8, 16)] = sms[c8] / cntf
            rowbuf[pl.ds(384 + 16 * c8, 16)] = sms[c8]
        pltpu.sync_copy(rowbuf, pooled_hbm.at[g])


def _final_mm_kernel(p_ref, w_ref, b_ref, o_ref):
    p = _leaky(p_ref[...], 0.01)
    o_ref[...] = jnp.dot(p, w_ref[...],
                         preferred_element_type=F32) + b_ref[...]


def kernel(x, edge_index, batch, Wl1, bl1, Wr1, br1, att1, bias1, Wl2, bl2,
           Wr2, br2, att2, bias2, Wl3, bl3, Wr3, br3, att3, bias3, Wh, bh):
    n = x.shape[0]
    e = edge_index.shape[1]
    npad = -(-n // ROWBLK) * ROWBLK
    eb = e + NB * PADQ
    G = 256

    src0, dst0 = edge_index[0], edge_index[1]
    srcb, dstb, hist = _bucket_edges(src0, dst0, eb)

    tot16 = hist.sum(0).astype(I32)
    sizes = (tot16 + PADQ - 1) // PADQ * PADQ
    bstart = (jnp.cumsum(sizes) - sizes).astype(I32)
    share = (tot16 + NT * 64 - 1) // (NT * 64) * 64
    bctl = jnp.stack([bstart, tot16, share]).astype(I32)

    xp = jnp.zeros((npad, 128), F32).at[:n, :8].set(x)
    w1 = jnp.zeros((128, 256), F32).at[:8].set(
        jnp.concatenate([Wl1, Wr1], axis=1))
    b1 = jnp.concatenate([bl1, br1])[None, :]
    slabs1 = _mm_slabs_1(xp, w1, b1, npad)
    alphaT1, amax1 = _run_passa(srcb, dstb, bctl, att1.reshape(-1), slabs1,
                                eb, npad, sum_heads=False)
    accv1, den1 = _run_passb(srcb, dstb, bctl, alphaT1, amax1, slabs1[:4],
                             eb, npad)

    w2 = jnp.concatenate([Wl2, Wr2], axis=1)
    b2 = jnp.concatenate([bl2, br2])[None, :]
    slabs2 = _mm_slabs_23(accv1, den1, bias1[None, :], w2, b2, npad)
    alphaT2, amax2 = _run_passa(srcb, dstb, bctl, att2.reshape(-1), slabs2,
                                eb, npad, sum_heads=False)
    accv2, den2 = _run_passb(srcb, dstb, bctl, alphaT2, amax2, slabs2[:4],
                             eb, npad)

    w3 = jnp.concatenate([Wl3, Wr3], axis=1)
    b3 = jnp.concatenate([bl3, br3])[None, :]
    slabs3 = _mm_slabs_23(accv2, den2, bias2[None, :], w3, b3, npad)
    alphaT3, amax3 = _run_passa(srcb, dstb, bctl, att3.reshape(-1), slabs3,
                                eb, npad, sum_heads=True)
    accv3, den3 = _run_passb(srcb, dstb, bctl, alphaT3, amax3, slabs3[:4],
                             eb, npad)
    h3 = _epilogue3(accv3, den3, bias3[None, :], npad)

    shp = npad // NT
    batchp = jnp.full((npad + 16,), G, I32).at[:n].set(batch)
    starts, ends = pl.kernel(
        functools.partial(_bnd_body, n, shp),
        out_type=[
            jax.ShapeDtypeStruct((264,), I32),
            jax.ShapeDtypeStruct((264,), I32),
        ],
        mesh=_mesh(),
        compiler_params=_SC_PARAMS,
        scratch_types=[
            pltpu.VMEM((shp + 16,), I32),
            pltpu.VMEM((16,), I32),
            pltpu.VMEM((16,), I32),
        ],
    )(batchp)

    pooled = pl.kernel(
        functools.partial(_pool_body, n, npad),
        out_type=jax.ShapeDtypeStruct((G, 512), F32),
        mesh=_mesh(),
        compiler_params=_SC_PARAMS,
        scratch_types=[
            pltpu.VMEM((264,), I32),
            pltpu.VMEM((264,), I32),
            pltpu.VMEM((64, 128), F32),
            pltpu.VMEM((512,), F32),
            pltpu.SemaphoreType.DMA,
        ],
    )(h3, starts, ends)

    out = pl.pallas_call(
        _final_mm_kernel,
        out_shape=jax.ShapeDtypeStruct((G, Wh.shape[1]), F32),
    )(pooled, Wh, bh[None, :])
    return out
